# baseline (speedup 1.0000x reference)
import jax
import jax.numpy as jnp
from jax.experimental import pallas as pl
from jax.experimental.pallas import tpu as pltpu

_PBLK = 128


def _body(ids_ref, gate_ref, hs_ref, emb_ref, tile_ref, out_ref):
    g = jnp.tanh(gate_ref[0])
    out_ref[...] = hs_ref[...] + (1.0 - g) * emb_ref[...] + g * tile_ref[...]


def kernel(hidden_state, aspect_ratio_ids, gate, embedding, tile_table):
    b, t, p, h = hidden_state.shape
    ids = aspect_ratio_ids.astype(jnp.int32)
    tile4 = tile_table.reshape(tile_table.shape[0], t, p, h)
    p_blocks = pl.cdiv(p, _PBLK)

    grid_spec = pltpu.PrefetchScalarGridSpec(
        num_scalar_prefetch=1,
        grid=(p_blocks, b),
        in_specs=[
            pl.BlockSpec(memory_space=pltpu.SMEM),
            pl.BlockSpec((1, t, _PBLK, h), lambda ip, ib, ids: (ib, 0, ip, 0)),
            pl.BlockSpec((_PBLK, h), lambda ip, ib, ids: (ip, 0)),
            pl.BlockSpec((1, t, _PBLK, h), lambda ip, ib, ids: (ids[ib], 0, ip, 0)),
        ],
        out_specs=pl.BlockSpec((1, t, _PBLK, h), lambda ip, ib, ids: (ib, 0, ip, 0)),
    )

    return pl.pallas_call(
        _body,
        grid_spec=grid_spec,
        out_shape=jax.ShapeDtypeStruct(hidden_state.shape, hidden_state.dtype),
    )(ids, gate, hidden_state, embedding, tile4)


# speedup vs baseline: 6.5654x; 6.5654x over previous
import jax
import jax.numpy as jnp
from jax.experimental import pallas as pl
from jax.experimental.pallas import tpu as pltpu


def _body(ids_ref, gate_ref, hs_ref, emb_ref, tile_ref, out_ref):
    p, h = emb_ref.shape
    g = jnp.tanh(gate_ref[0])
    tile = tile_ref[...].reshape(p, h)
    out_ref[...] = hs_ref[...] + ((1.0 - g) * emb_ref[...] + g * tile)[None, None]


def kernel(hidden_state, aspect_ratio_ids, gate, embedding, tile_table):
    b, t, p, h = hidden_state.shape
    ids = aspect_ratio_ids.astype(jnp.int32)
    ph = p * h

    grid_spec = pltpu.PrefetchScalarGridSpec(
        num_scalar_prefetch=1,
        grid=(t, b),
        in_specs=[
            pl.BlockSpec(memory_space=pltpu.SMEM),
            pl.BlockSpec((1, 1, p, h), lambda it, ib, ids: (ib, it, 0, 0)),
            pl.BlockSpec((p, h), lambda it, ib, ids: (0, 0)),
            pl.BlockSpec((1, 1, ph), lambda it, ib, ids: (ids[ib], 0, it)),
        ],
        out_specs=pl.BlockSpec((1, 1, p, h), lambda it, ib, ids: (ib, it, 0, 0)),
    )

    return pl.pallas_call(
        _body,
        grid_spec=grid_spec,
        out_shape=jax.ShapeDtypeStruct(hidden_state.shape, hidden_state.dtype),
    )(ids, gate, hidden_state, embedding, tile_table[:, None, :])


# speedup vs baseline: 6.5860x; 1.0031x over previous
import jax
import jax.numpy as jnp
from jax.experimental import pallas as pl
from jax.experimental.pallas import tpu as pltpu


def _body(scal_ref, gate_ref, hs_ref, emb_ref, tile_ref, out_ref):
    p, h = emb_ref.shape
    g = jnp.tanh(gate_ref[0])
    tile = tile_ref[...].reshape(p, h)
    out_ref[...] = hs_ref[...] + ((1.0 - g) * emb_ref[...] + g * tile)[None, None]


def kernel(hidden_state, aspect_ratio_ids, gate, embedding, tile_table):
    b, t, p, h = hidden_state.shape
    ids = aspect_ratio_ids.astype(jnp.int32)
    perm = jnp.argsort(ids).astype(jnp.int32)
    sids = jnp.take(ids, perm)
    scal = jnp.concatenate([sids, perm])
    ph = p * h

    grid_spec = pltpu.PrefetchScalarGridSpec(
        num_scalar_prefetch=1,
        grid=(t, b),
        in_specs=[
            pl.BlockSpec(memory_space=pltpu.SMEM),
            pl.BlockSpec((1, 1, p, h), lambda it, ik, s: (s[b + ik], it, 0, 0)),
            pl.BlockSpec((p, h), lambda it, ik, s: (0, 0)),
            pl.BlockSpec((1, 1, ph), lambda it, ik, s: (s[ik], 0, it)),
        ],
        out_specs=pl.BlockSpec((1, 1, p, h), lambda it, ik, s: (s[b + ik], it, 0, 0)),
    )

    return pl.pallas_call(
        _body,
        grid_spec=grid_spec,
        out_shape=jax.ShapeDtypeStruct(hidden_state.shape, hidden_state.dtype),
    )(scal, gate, hidden_state, embedding, tile_table[:, None, :])
